# Initial kernel scaffold; baseline (speedup 1.0000x reference)
import jax
import jax.numpy as jnp
from jax.experimental import pallas as pl


def kernel(sem_feats, boxes_per_cls, obj_labels, W_out, b_out, obj_embed_weight):
    raise NotImplementedError("write your pallas kernel here")


# speedup vs baseline: 9.7852x; 9.7852x over previous
import jax
import jax.numpy as jnp
from jax import lax
from jax.experimental import pallas as pl

NUM_CLS = 151
C_PAD = 152
EMBED_DIM = 200
HIDDEN = 512
N_PER = 256
N_IMG = 4
NMS_THRESH = 0.5


def _decode_kernel(featsT_ref, wT_ref, b_ref, geom_ref, tableT_ref,
                   distsT_ref, labels_ref, embedT_ref):
    row_iota = lax.broadcasted_iota(jnp.int32, (C_PAD, N_PER), 0)
    lane_iota = lax.broadcasted_iota(jnp.int32, (C_PAD, N_PER), 1)
    lane1 = lax.broadcasted_iota(jnp.int32, (1, N_PER), 1)

    dT = jnp.dot(wT_ref[...], featsT_ref[0],
                 preferred_element_type=jnp.float32) + b_ref[...]
    dT = jnp.where(row_iota >= NUM_CLS, -1e30, dT)
    distsT_ref[0] = dT

    m = jnp.max(dT, axis=0, keepdims=True)
    e = jnp.exp(dT - m)
    p = e / jnp.sum(e, axis=0, keepdims=True)
    p = jnp.where(row_iota == 0, -1.0, p)
    p = jnp.where(row_iota >= NUM_CLS, -1e9, p)

    big = jnp.int32(1 << 30)

    def body(_, carry):
        p, labels = carry
        mx = jnp.max(p)
        is_mx = p == mx
        box = jnp.min(jnp.where(is_mx, lane_iota, big))
        cls = jnp.min(jnp.where(is_mx & (lane_iota == box), row_iota, big))

        g = geom_ref[0, cls]
        x1 = g[0:1]
        y1 = g[1:2]
        x2 = g[2:3]
        y2 = g[3:4]
        onb = lane1 == box
        px1 = jnp.sum(jnp.where(onb, x1, 0.0))
        py1 = jnp.sum(jnp.where(onb, y1, 0.0))
        px2 = jnp.sum(jnp.where(onb, x2, 0.0))
        py2 = jnp.sum(jnp.where(onb, y2, 0.0))

        iw = jnp.maximum(jnp.minimum(px2, x2) - jnp.maximum(px1, x1) + 1.0, 0.0)
        ih = jnp.maximum(jnp.minimum(py2, y2) - jnp.maximum(py1, y1) + 1.0, 0.0)
        inter = iw * ih
        areas = (x2 - x1 + 1.0) * (y2 - y1 + 1.0)
        parea = (px2 - px1 + 1.0) * (py2 - py1 + 1.0)
        iou = inter / (parea + areas - inter)
        mask = iou >= NMS_THRESH

        p = jnp.where((row_iota == cls) & mask, 0.0, p)
        p = jnp.where(lane_iota == box, -1.0, p)
        labels = jnp.where(onb, cls, labels)
        return p, labels

    labels0 = jnp.zeros((1, N_PER), jnp.int32)
    _, labels = lax.fori_loop(0, N_PER, body, (p, labels0))
    labels_ref[0] = labels

    onehotT = (row_iota == labels).astype(jnp.float32)
    embedT_ref[0] = jnp.dot(tableT_ref[...], onehotT,
                            preferred_element_type=jnp.float32,
                            precision=lax.Precision.HIGHEST)


def kernel(sem_feats, boxes_per_cls, obj_labels, W_out, b_out, obj_embed_weight):
    del obj_labels
    n_total = N_IMG * N_PER
    featsT = sem_feats.reshape(N_IMG, N_PER, HIDDEN).transpose(0, 2, 1)
    wT = jnp.pad(W_out.T, ((0, C_PAD - NUM_CLS), (0, 0)))
    b_col = jnp.pad(b_out, (0, C_PAD - NUM_CLS)).reshape(C_PAD, 1)
    geom = boxes_per_cls.reshape(N_IMG, N_PER, NUM_CLS, 4).transpose(0, 2, 3, 1)
    geom = jnp.pad(geom, ((0, 0), (0, C_PAD - NUM_CLS), (0, 0), (0, 0)))
    tableT = jnp.pad(obj_embed_weight.T, ((0, 0), (0, C_PAD - NUM_CLS)))

    distsT, labels, embedT = pl.pallas_call(
        _decode_kernel,
        grid=(N_IMG,),
        in_specs=[
            pl.BlockSpec((1, HIDDEN, N_PER), lambda i: (i, 0, 0)),
            pl.BlockSpec((C_PAD, HIDDEN), lambda i: (0, 0)),
            pl.BlockSpec((C_PAD, 1), lambda i: (0, 0)),
            pl.BlockSpec((1, C_PAD, 4, N_PER), lambda i: (i, 0, 0, 0)),
            pl.BlockSpec((EMBED_DIM, C_PAD), lambda i: (0, 0)),
        ],
        out_specs=[
            pl.BlockSpec((1, C_PAD, N_PER), lambda i: (i, 0, 0)),
            pl.BlockSpec((1, 1, N_PER), lambda i: (i, 0, 0)),
            pl.BlockSpec((1, EMBED_DIM, N_PER), lambda i: (i, 0, 0)),
        ],
        out_shape=[
            jax.ShapeDtypeStruct((N_IMG, C_PAD, N_PER), jnp.float32),
            jax.ShapeDtypeStruct((N_IMG, 1, N_PER), jnp.int32),
            jax.ShapeDtypeStruct((N_IMG, EMBED_DIM, N_PER), jnp.float32),
        ],
    )(featsT, wT, b_col, geom, tableT)

    obj_dists = distsT[:, :NUM_CLS, :].transpose(0, 2, 1).reshape(n_total, NUM_CLS)
    obj_preds = labels.reshape(n_total)
    obj_embed_out = embedT.transpose(0, 2, 1).reshape(n_total, EMBED_DIM)
    return (obj_dists, obj_preds, obj_embed_out)


# speedup vs baseline: 44.4614x; 4.5438x over previous
import jax
import jax.numpy as jnp
from jax import lax
from jax.experimental import pallas as pl

NUM_CLS = 151
C_PAD = 152
EMBED_DIM = 200
HIDDEN = 512
N_PER = 256
N_IMG = 4
NMS_THRESH = 0.5


def _decode_kernel(featsT_ref, wT_ref, b_ref, geom_ref, tableT_ref,
                   distsT_ref, labels_ref, embedT_ref):
    row_iota = lax.broadcasted_iota(jnp.int32, (C_PAD, N_PER), 0)
    lane_iota = lax.broadcasted_iota(jnp.int32, (C_PAD, N_PER), 1)
    lane1 = lax.broadcasted_iota(jnp.int32, (1, N_PER), 1)
    packed = lane_iota * C_PAD + row_iota
    big = jnp.int32(1 << 30)

    ps = []
    for i in range(N_IMG):
        dT = jnp.dot(wT_ref[...], featsT_ref[i],
                     preferred_element_type=jnp.float32) + b_ref[...]
        dT = jnp.where(row_iota >= NUM_CLS, -1e30, dT)
        distsT_ref[i] = dT
        m = jnp.max(dT, axis=0, keepdims=True)
        e = jnp.exp(dT - m)
        p = e / jnp.sum(e, axis=0, keepdims=True)
        p = jnp.where(row_iota == 0, -1.0, p)
        p = jnp.where(row_iota >= NUM_CLS, -1e9, p)
        ps.append(p)

    def body(_, carry):
        out = []
        for i in range(N_IMG):
            p, labels = carry[i]
            mx = jnp.max(jnp.max(p, axis=0, keepdims=True),
                         axis=1, keepdims=True)
            pk = jnp.min(jnp.where(p == mx, packed, big))
            box = ((pk.astype(jnp.float32) + 0.5)
                   * (1.0 / C_PAD)).astype(jnp.int32)
            cls = pk - box * C_PAD

            g = geom_ref[i, cls]
            x1 = g[0:1]
            y1 = g[1:2]
            x2 = g[2:3]
            y2 = g[3:4]
            onb = lane1 == box
            px1 = jnp.sum(jnp.where(onb, x1, 0.0), axis=1, keepdims=True)
            py1 = jnp.sum(jnp.where(onb, y1, 0.0), axis=1, keepdims=True)
            px2 = jnp.sum(jnp.where(onb, x2, 0.0), axis=1, keepdims=True)
            py2 = jnp.sum(jnp.where(onb, y2, 0.0), axis=1, keepdims=True)

            iw = jnp.maximum(jnp.minimum(px2, x2) - jnp.maximum(px1, x1) + 1.0, 0.0)
            ih = jnp.maximum(jnp.minimum(py2, y2) - jnp.maximum(py1, y1) + 1.0, 0.0)
            inter = iw * ih
            areas = (x2 - x1 + 1.0) * (y2 - y1 + 1.0)
            parea = (px2 - px1 + 1.0) * (py2 - py1 + 1.0)
            iou = inter / (parea + areas - inter)
            mask = iou >= NMS_THRESH

            p = jnp.where((row_iota == cls) & mask, 0.0, p)
            p = jnp.where(lane_iota == box, -1.0, p)
            labels = jnp.where(onb, cls, labels)
            out.append((p, labels))
        return tuple(out)

    carry0 = tuple((p, jnp.zeros((1, N_PER), jnp.int32)) for p in ps)
    carry = lax.fori_loop(0, N_PER, body, carry0)

    for i in range(N_IMG):
        labels = carry[i][1]
        labels_ref[i] = labels
        onehotT = (row_iota == labels).astype(jnp.float32)
        embedT_ref[i] = jnp.dot(tableT_ref[...], onehotT,
                                preferred_element_type=jnp.float32,
                                precision=lax.Precision.HIGHEST)


def kernel(sem_feats, boxes_per_cls, obj_labels, W_out, b_out, obj_embed_weight):
    del obj_labels
    n_total = N_IMG * N_PER
    featsT = sem_feats.reshape(N_IMG, N_PER, HIDDEN).transpose(0, 2, 1)
    wT = jnp.pad(W_out.T, ((0, C_PAD - NUM_CLS), (0, 0)))
    b_col = jnp.pad(b_out, (0, C_PAD - NUM_CLS)).reshape(C_PAD, 1)
    geom = boxes_per_cls.reshape(N_IMG, N_PER, NUM_CLS, 4).transpose(0, 2, 3, 1)
    geom = jnp.pad(geom, ((0, 0), (0, C_PAD - NUM_CLS), (0, 0), (0, 0)))
    tableT = jnp.pad(obj_embed_weight.T, ((0, 0), (0, C_PAD - NUM_CLS)))

    distsT, labels, embedT = pl.pallas_call(
        _decode_kernel,
        out_shape=[
            jax.ShapeDtypeStruct((N_IMG, C_PAD, N_PER), jnp.float32),
            jax.ShapeDtypeStruct((N_IMG, 1, N_PER), jnp.int32),
            jax.ShapeDtypeStruct((N_IMG, EMBED_DIM, N_PER), jnp.float32),
        ],
    )(featsT, wT, b_col, geom, tableT)

    obj_dists = distsT[:, :NUM_CLS, :].transpose(0, 2, 1).reshape(n_total, NUM_CLS)
    obj_preds = labels.reshape(n_total)
    obj_embed_out = embedT.transpose(0, 2, 1).reshape(n_total, EMBED_DIM)
    return (obj_dists, obj_preds, obj_embed_out)
